# baseline (speedup 1.0000x reference)
import functools

import jax
import jax.numpy as jnp
from jax import lax
from jax.experimental import pallas as pl
from jax.experimental.pallas import tpu as pltpu
from jax.experimental.pallas import tpu_sc as plsc

_NC = 2
_NS = 16
_L = 16


def _make_agg(N, D, E, with_deg, NP):
    NW = _NC * _NS
    C = 80
    EPW = -(-(E // NW) // C) * C
    NCH = EPW // C
    PAD = EPW - E // NW
    NA = N + max(8, PAD)
    FULL = NP // _NS
    LAST = N - FULL * (_NS - 1)
    ZC = C
    mesh = plsc.VectorSubcoreMesh(core_axis_name="c", subcore_axis_name="s")

    out_type = [jax.ShapeDtypeStruct((_NC, N, D), jnp.float32)]
    scratch = [
        pltpu.VMEM_SHARED((NA, D), jnp.float32),
        pltpu.VMEM((4, C), jnp.int32),
        pltpu.VMEM((NCH, C), jnp.int32),
        [pltpu.VMEM((C, D), jnp.float32) for _ in range(3)],
        [pltpu.SemaphoreType.DMA for _ in range(3)],
        [pltpu.SemaphoreType.DMA for _ in range(3)],
        pltpu.SemaphoreType.DMA((4,)),
        pltpu.SemaphoreType.DMA,
    ]
    if with_deg:
        out_type.append(jax.ShapeDtypeStruct((_NC * N,), jnp.float32))
        scratch += [
            pltpu.VMEM_SHARED((NA,), jnp.float32),
            pltpu.VMEM((C,), jnp.float32),
            pltpu.VMEM((FULL,), jnp.float32),
        ]

    @functools.partial(pl.kernel, out_type=out_type, mesh=mesh,
                       scratch_types=scratch)
    def agg(*refs):
        if with_deg:
            (y_hbm, src_hbm, dst_hbm, out_hbm, deg_hbm,
             acc, srcb, dstv, rows, gsems, ssems, isems, dsem,
             dacc, ones, dzero) = refs
        else:
            (y_hbm, src_hbm, dst_hbm, out_hbm,
             acc, srcb, dstv, rows, gsems, ssems, isems, dsem) = refs
        c = lax.axis_index("c")
        s = lax.axis_index("s")
        wid = s * _NC + c
        ebase = wid * EPW
        zv = jnp.zeros((_L,), jnp.float32)

        def idx_desc(g, q):
            off = pl.multiple_of(ebase + g * C, 8)
            return pltpu.make_async_copy(src_hbm.at[pl.ds(off, C)],
                                         srcb.at[q], isems.at[q])

        def gather_desc(g, b):
            q = g % 4 if isinstance(g, int) else lax.rem(g, 4)
            return pltpu.make_async_copy(y_hbm.at[srcb.at[q]], rows[b],
                                         gsems[b])

        for q in range(4):
            idx_desc(q, q).start()
        dslab = pltpu.make_async_copy(dst_hbm.at[wid], dstv, dsem)
        dslab.start()

        def zrow(i, carry):
            for b in range(3):
                for j in range(D // _L):
                    rows[b][i, pl.ds(j * _L, _L)] = zv
            return carry
        lax.fori_loop(0, C, zrow, 0)
        n_last = LAST // ZC
        rem = LAST % ZC
        for k in range(FULL // ZC):
            def zcopy(nr=ZC, k=k):
                pltpu.sync_copy(rows[k % 3].at[pl.ds(0, nr)],
                                acc.at[pl.ds(s * FULL + k * ZC, nr)])
            if k < n_last:
                zcopy()
            else:
                pl.when(s < _NS - 1)(zcopy)
                if k == n_last and rem:
                    pl.when(s == _NS - 1)(lambda: zcopy(rem))
        if with_deg:
            ov = jnp.full((_L,), 1.0, jnp.float32)
            for i in range(C // _L):
                ones[pl.ds(i * _L, _L)] = ov
            def dzrow(i, carry):
                dzero[pl.ds(i * _L, _L)] = zv
                return carry
            lax.fori_loop(0, FULL // _L, dzrow, 0)
            @pl.when(s < _NS - 1)
            def _():
                pltpu.sync_copy(dzero, dacc.at[pl.ds(s * FULL, FULL)])
            @pl.when(s == _NS - 1)
            def _():
                pltpu.sync_copy(dzero.at[pl.ds(0, LAST)],
                                dacc.at[pl.ds(s * FULL, LAST)])
        plsc.subcore_barrier()

        idx_desc(0, 0).wait()
        gather_desc(0, 0).start()
        idx_desc(1, 1).wait()
        gather_desc(1, 1).start()
        dslab.wait()

        def scat_desc(g, b):
            return pltpu.make_async_copy(rows[b], acc.at[dstv.at[g]],
                                         ssems[b])

        def step(g, b):
            gather_desc(g, b).wait()
            @pl.when(g >= 1)
            def _():
                scat_desc(g - 1, (b + 2) % 3).wait()
            @pl.when(g + 2 < NCH)
            def _():
                idx_desc(g + 2, lax.rem(g + 2, 4)).wait()
                gather_desc(g + 2, (b + 2) % 3).start()
            pltpu.async_copy(rows[b], acc.at[dstv.at[g]], ssems[b], add=True)
            if with_deg:
                pltpu.sync_copy(ones, dacc.at[dstv.at[g]], add=True)
            @pl.when(g + 4 < NCH)
            def _():
                idx_desc(g + 4, lax.rem(g, 4)).start()

        def grp(gg, carry):
            g0 = gg * 3
            for b in range(3):
                @pl.when(g0 + b < NCH)
                def _(b=b):
                    step(g0 + b, b)
            return carry
        lax.fori_loop(0, (NCH + 2) // 3, grp, 0)
        scat_desc(NCH - 1, (NCH - 1) % 3).wait()

        plsc.subcore_barrier()
        @pl.when(s < _NS - 1)
        def _():
            pltpu.sync_copy(acc.at[pl.ds(s * FULL, FULL)],
                            out_hbm.at[c, pl.ds(s * FULL, FULL)])
        @pl.when(s == _NS - 1)
        def _():
            pltpu.sync_copy(acc.at[pl.ds(s * FULL, LAST)],
                            out_hbm.at[c, pl.ds(s * FULL, LAST)])
        if with_deg:
            @pl.when(s < _NS - 1)
            def _():
                pltpu.sync_copy(dacc.at[pl.ds(s * FULL, FULL)], dzero)
                pltpu.sync_copy(
                    dzero, deg_hbm.at[pl.ds(c * N + s * FULL, FULL)])
            @pl.when(s == _NS - 1)
            def _():
                pltpu.sync_copy(dacc.at[pl.ds(s * FULL, LAST)],
                                dzero.at[pl.ds(0, LAST)])
                pltpu.sync_copy(
                    dzero.at[pl.ds(0, LAST)],
                    deg_hbm.at[pl.ds(c * N + s * FULL, LAST)])

    return agg


def _make_deg(N, E, NP):
    NW = _NC * _NS
    C = 80
    EPW = -(-(E // NW) // C) * C
    NCH = EPW // C
    PAD = EPW - E // NW
    NA = N + max(8, PAD)
    FULL = NP // _NS
    LAST = N - FULL * (_NS - 1)
    mesh = plsc.VectorSubcoreMesh(core_axis_name="c", subcore_axis_name="s")

    @functools.partial(
        pl.kernel,
        out_type=jax.ShapeDtypeStruct((_NC * N,), jnp.float32),
        mesh=mesh,
        scratch_types=[
            pltpu.VMEM_SHARED((NA,), jnp.float32),
            pltpu.VMEM((NCH, C), jnp.int32),
            pltpu.VMEM((C,), jnp.float32),
            pltpu.VMEM((FULL,), jnp.float32),
            pltpu.SemaphoreType.DMA,
            pltpu.SemaphoreType.DMA,
        ])
    def deg(dst_hbm, deg_hbm, dacc, dstv, ones, dzero, dsem, ssem):
        c = lax.axis_index("c")
        s = lax.axis_index("s")
        wid = s * _NC + c
        zv = jnp.zeros((_L,), jnp.float32)
        dslab = pltpu.make_async_copy(dst_hbm.at[wid], dstv, dsem)
        dslab.start()
        ov = jnp.full((_L,), 1.0, jnp.float32)
        for i in range(C // _L):
            ones[pl.ds(i * _L, _L)] = ov
        def dzrow(i, carry):
            dzero[pl.ds(i * _L, _L)] = zv
            return carry
        lax.fori_loop(0, FULL // _L, dzrow, 0)
        @pl.when(s < _NS - 1)
        def _():
            pltpu.sync_copy(dzero, dacc.at[pl.ds(s * FULL, FULL)])
        @pl.when(s == _NS - 1)
        def _():
            pltpu.sync_copy(dzero.at[pl.ds(0, LAST)],
                            dacc.at[pl.ds(s * FULL, LAST)])
        dslab.wait()
        plsc.subcore_barrier()

        B = 5
        def grp(gg, carry):
            g0 = gg * B
            for j in range(B):
                pltpu.async_copy(ones, dacc.at[dstv.at[g0 + j]], ssem,
                                 add=True)
            for j in range(B):
                pltpu.make_async_copy(ones, dacc.at[dstv.at[g0 + j]],
                                      ssem).wait()
            return carry
        lax.fori_loop(0, NCH // B, grp, 0)
        for g in range(NCH - NCH % B, NCH):
            pltpu.sync_copy(ones, dacc.at[dstv.at[g]], add=True)

        plsc.subcore_barrier()
        @pl.when(s < _NS - 1)
        def _():
            pltpu.sync_copy(dacc.at[pl.ds(s * FULL, FULL)], dzero)
            pltpu.sync_copy(dzero, deg_hbm.at[pl.ds(c * N + s * FULL, FULL)])
        @pl.when(s == _NS - 1)
        def _():
            pltpu.sync_copy(dacc.at[pl.ds(s * FULL, LAST)],
                            dzero.at[pl.ds(0, LAST)])
            pltpu.sync_copy(dzero.at[pl.ds(0, LAST)],
                            deg_hbm.at[pl.ds(c * N + s * FULL, LAST)])

    return deg


def _proj(x, Wl, Wr, bl):
    N, Din = x.shape
    Do = Wl.shape[1]
    BN = 1000

    def body(x_ref, wl_ref, wr_ref, b_ref, y_ref, z_ref):
        xb = x_ref[...]
        y_ref[...] = jnp.dot(xb, wl_ref[...],
                             preferred_element_type=jnp.float32)
        z_ref[...] = jnp.dot(xb, wr_ref[...],
                             preferred_element_type=jnp.float32) + b_ref[...]

    y, z = pl.pallas_call(
        body,
        grid=(N // BN,),
        in_specs=[
            pl.BlockSpec((BN, Din), lambda i: (i, 0)),
            pl.BlockSpec((Din, Do), lambda i: (0, 0)),
            pl.BlockSpec((Din, Do), lambda i: (0, 0)),
            pl.BlockSpec((1, Do), lambda i: (0, 0)),
        ],
        out_specs=[
            pl.BlockSpec((BN, Do), lambda i: (i, 0)),
            pl.BlockSpec((BN, Do), lambda i: (i, 0)),
        ],
        out_shape=[jax.ShapeDtypeStruct((N, Do), jnp.float32)] * 2,
    )(x, Wl, Wr, bl.reshape(1, -1))
    return y, z


def _comb_proj(p0, p1, z, d0, d1, Wl, Wr, bl, emit_h):
    N, Dh = z.shape
    Do = Wr.shape[1]
    BN = 1000

    def body(p0_ref, p1_ref, z_ref, d0_ref, d1_ref, wl_ref, wr_ref, b_ref,
             y_ref, z2_ref):
        deg = jnp.maximum(d0_ref[...] + d1_ref[...], 1.0)
        h = jnp.maximum((p0_ref[...] + p1_ref[...]) / deg + z_ref[...], 0.0)
        if emit_h:
            y_ref[...] = h
        else:
            y_ref[...] = jnp.dot(h, wl_ref[...],
                                 preferred_element_type=jnp.float32)
        z2_ref[...] = jnp.dot(h, wr_ref[...],
                              preferred_element_type=jnp.float32) + b_ref[...]

    return pl.pallas_call(
        body,
        grid=(N // BN,),
        in_specs=[
            pl.BlockSpec((BN, Dh), lambda i: (i, 0)),
            pl.BlockSpec((BN, Dh), lambda i: (i, 0)),
            pl.BlockSpec((BN, Dh), lambda i: (i, 0)),
            pl.BlockSpec((BN, 1), lambda i: (i, 0)),
            pl.BlockSpec((BN, 1), lambda i: (i, 0)),
            pl.BlockSpec(Wl.shape, lambda i: (0, 0)),
            pl.BlockSpec((Dh, Do), lambda i: (0, 0)),
            pl.BlockSpec((1, Do), lambda i: (0, 0)),
        ],
        out_specs=[
            pl.BlockSpec((BN, Dh), lambda i: (i, 0)),
            pl.BlockSpec((BN, Do), lambda i: (i, 0)),
        ],
        out_shape=[jax.ShapeDtypeStruct((N, Dh), jnp.float32),
                   jax.ShapeDtypeStruct((N, Do), jnp.float32)],
    )(p0, p1, z, d0, d1, Wl, Wr, bl.reshape(1, -1))


def _final(p0, p1, z2, d0, d1, Wl):
    N, Dh = p0.shape
    Do = Wl.shape[1]
    BN = 1000

    def body(p0_ref, p1_ref, z2_ref, d0_ref, d1_ref, wl_ref, o_ref):
        deg = jnp.maximum(d0_ref[...] + d1_ref[...], 1.0)
        m = (p0_ref[...] + p1_ref[...]) / deg
        u = (jnp.dot(m, wl_ref[...], preferred_element_type=jnp.float32)
             + z2_ref[...])
        mx = jnp.max(u, axis=1, keepdims=True)
        e = u - mx
        o_ref[...] = e - jnp.log(jnp.sum(jnp.exp(e), axis=1, keepdims=True))

    return pl.pallas_call(
        body,
        grid=(N // BN,),
        in_specs=[
            pl.BlockSpec((BN, Dh), lambda i: (i, 0)),
            pl.BlockSpec((BN, Dh), lambda i: (i, 0)),
            pl.BlockSpec((BN, Do), lambda i: (i, 0)),
            pl.BlockSpec((BN, 1), lambda i: (i, 0)),
            pl.BlockSpec((BN, 1), lambda i: (i, 0)),
            pl.BlockSpec((Dh, Do), lambda i: (0, 0)),
        ],
        out_specs=pl.BlockSpec((BN, Do), lambda i: (i, 0)),
        out_shape=jax.ShapeDtypeStruct((N, Do), jnp.float32),
    )(p0, p1, z2, d0, d1, Wl)


def kernel(x, edge_index, Wl0, bl0, Wr0, Wl1, bl1, Wr1, Wl2, bl2, Wr2):
    N, Din = x.shape
    E = edge_index.shape[1]
    Dh = Wl0.shape[1]
    NP = ((N + 128 * _NS - 1) // (128 * _NS)) * (128 * _NS)
    NW = _NC * _NS
    C = 80
    EPW_r = E // NW
    EPW = -(-EPW_r // C) * C
    pad = EPW - EPW_r
    src = jnp.pad(edge_index[0].reshape(NW, EPW_r),
                  ((0, 0), (0, pad))).reshape(-1)
    dump = jnp.broadcast_to(N + jnp.arange(pad, dtype=jnp.int32),
                            (NW, pad))
    dst = jnp.concatenate(
        [edge_index[1].reshape(NW, EPW_r), dump],
        axis=1).reshape(NW, EPW // C, C)

    agg_h = _make_agg(N, Dh, E, False, NP)

    degf = _make_deg(N, E, NP)(dst)
    y, z = _proj(x, Wl0, Wr0, bl0)
    (p,) = agg_h(y, src, dst)
    degp = degf.reshape(_NC, N)
    d0 = degp[0].reshape(N, 1)
    d1 = degp[1].reshape(N, 1)

    y, z = _comb_proj(p[0], p[1], z, d0, d1, Wl1, Wr1, bl1, emit_h=False)
    (p,) = agg_h(y, src, dst)

    h2, z2 = _comb_proj(p[0], p[1], z, d0, d1, Wl2, Wr2, bl2, emit_h=True)
    (p,) = agg_h(h2, src, dst)
    return _final(p[0], p[1], z2, d0, d1, Wl2)


# speedup vs baseline: 1.0126x; 1.0126x over previous
import functools

import jax
import jax.numpy as jnp
from jax import lax
from jax.experimental import pallas as pl
from jax.experimental.pallas import tpu as pltpu
from jax.experimental.pallas import tpu_sc as plsc

_NC = 2
_NS = 16
_L = 16


def _make_agg(N, D, E, with_deg, NP):
    NW = _NC * _NS
    C = 80
    EPW = -(-(E // NW) // C) * C
    NCH = EPW // C
    PAD = EPW - E // NW
    NA = N + max(8, PAD)
    FULL = NP // _NS
    LAST = N - FULL * (_NS - 1)
    ZC = C
    mesh = plsc.VectorSubcoreMesh(core_axis_name="c", subcore_axis_name="s")

    out_type = [jax.ShapeDtypeStruct((_NC, N, D), jnp.float32)]
    scratch = [
        pltpu.VMEM_SHARED((NA, D), jnp.float32),
        pltpu.VMEM((4, C), jnp.int32),
        pltpu.VMEM((NCH, C), jnp.int32),
        [pltpu.VMEM((C, D), jnp.float32) for _ in range(3)],
        [pltpu.SemaphoreType.DMA for _ in range(3)],
        [pltpu.SemaphoreType.DMA for _ in range(3)],
        pltpu.SemaphoreType.DMA((4,)),
        pltpu.SemaphoreType.DMA,
    ]
    if with_deg:
        out_type.append(jax.ShapeDtypeStruct((_NC * N,), jnp.float32))
        scratch += [
            pltpu.VMEM_SHARED((NA,), jnp.float32),
            pltpu.VMEM((C,), jnp.float32),
            pltpu.VMEM((FULL,), jnp.float32),
        ]

    @functools.partial(pl.kernel, out_type=out_type, mesh=mesh,
                       scratch_types=scratch)
    def agg(*refs):
        if with_deg:
            (y_hbm, src_hbm, dst_hbm, out_hbm, deg_hbm,
             acc, srcb, dstv, rows, gsems, ssems, isems, dsem,
             dacc, ones, dzero) = refs
        else:
            (y_hbm, src_hbm, dst_hbm, out_hbm,
             acc, srcb, dstv, rows, gsems, ssems, isems, dsem) = refs
        c = lax.axis_index("c")
        s = lax.axis_index("s")
        wid = s * _NC + c
        ebase = wid * EPW
        zv = jnp.zeros((_L,), jnp.float32)

        def idx_desc(g, q):
            off = pl.multiple_of(ebase + g * C, 8)
            return pltpu.make_async_copy(src_hbm.at[pl.ds(off, C)],
                                         srcb.at[q], isems.at[q])

        def gather_desc(g, b):
            q = g % 4 if isinstance(g, int) else lax.rem(g, 4)
            return pltpu.make_async_copy(y_hbm.at[srcb.at[q]], rows[b],
                                         gsems[b])

        for q in range(4):
            idx_desc(q, q).start()
        dslab = pltpu.make_async_copy(dst_hbm.at[wid], dstv, dsem)
        dslab.start()

        def zrow(i, carry):
            for b in range(3):
                for j in range(D // _L):
                    rows[b][i, pl.ds(j * _L, _L)] = zv
            return carry
        lax.fori_loop(0, C, zrow, 0)
        n_last = LAST // ZC
        rem = LAST % ZC
        for k in range(FULL // ZC):
            def zcopy(nr=ZC, k=k):
                pltpu.sync_copy(rows[k % 3].at[pl.ds(0, nr)],
                                acc.at[pl.ds(s * FULL + k * ZC, nr)])
            if k < n_last:
                zcopy()
            else:
                pl.when(s < _NS - 1)(zcopy)
                if k == n_last and rem:
                    pl.when(s == _NS - 1)(lambda: zcopy(rem))
        if with_deg:
            ov = jnp.full((_L,), 1.0, jnp.float32)
            for i in range(C // _L):
                ones[pl.ds(i * _L, _L)] = ov
            def dzrow(i, carry):
                dzero[pl.ds(i * _L, _L)] = zv
                return carry
            lax.fori_loop(0, FULL // _L, dzrow, 0)
            @pl.when(s < _NS - 1)
            def _():
                pltpu.sync_copy(dzero, dacc.at[pl.ds(s * FULL, FULL)])
            @pl.when(s == _NS - 1)
            def _():
                pltpu.sync_copy(dzero.at[pl.ds(0, LAST)],
                                dacc.at[pl.ds(s * FULL, LAST)])
        plsc.subcore_barrier()

        idx_desc(0, 0).wait()
        gather_desc(0, 0).start()
        idx_desc(1, 1).wait()
        gather_desc(1, 1).start()
        dslab.wait()

        def scat_desc(g, b):
            return pltpu.make_async_copy(rows[b], acc.at[dstv.at[g]],
                                         ssems[b])

        def step(g, b):
            gather_desc(g, b).wait()
            @pl.when(g >= 1)
            def _():
                scat_desc(g - 1, (b + 2) % 3).wait()
            @pl.when(g + 2 < NCH)
            def _():
                idx_desc(g + 2, lax.rem(g + 2, 4)).wait()
                gather_desc(g + 2, (b + 2) % 3).start()
            pltpu.async_copy(rows[b], acc.at[dstv.at[g]], ssems[b], add=True)
            if with_deg:
                pltpu.sync_copy(ones, dacc.at[dstv.at[g]], add=True)
            @pl.when(g + 4 < NCH)
            def _():
                idx_desc(g + 4, lax.rem(g, 4)).start()

        def grp(gg, carry):
            g0 = gg * 3
            for b in range(3):
                @pl.when(g0 + b < NCH)
                def _(b=b):
                    step(g0 + b, b)
            return carry
        lax.fori_loop(0, (NCH + 2) // 3, grp, 0)
        scat_desc(NCH - 1, (NCH - 1) % 3).wait()

        plsc.subcore_barrier()
        @pl.when(s < _NS - 1)
        def _():
            pltpu.sync_copy(acc.at[pl.ds(s * FULL, FULL)],
                            out_hbm.at[c, pl.ds(s * FULL, FULL)])
        @pl.when(s == _NS - 1)
        def _():
            pltpu.sync_copy(acc.at[pl.ds(s * FULL, LAST)],
                            out_hbm.at[c, pl.ds(s * FULL, LAST)])
        if with_deg:
            @pl.when(s < _NS - 1)
            def _():
                pltpu.sync_copy(dacc.at[pl.ds(s * FULL, FULL)], dzero)
                pltpu.sync_copy(
                    dzero, deg_hbm.at[pl.ds(c * N + s * FULL, FULL)])
            @pl.when(s == _NS - 1)
            def _():
                pltpu.sync_copy(dacc.at[pl.ds(s * FULL, LAST)],
                                dzero.at[pl.ds(0, LAST)])
                pltpu.sync_copy(
                    dzero.at[pl.ds(0, LAST)],
                    deg_hbm.at[pl.ds(c * N + s * FULL, LAST)])

    return agg


def _make_deg(N, E, NP):
    NW = _NC * _NS
    C = 80
    EPW = -(-(E // NW) // C) * C
    NCH = EPW // C
    PAD = EPW - E // NW
    NA = N + max(8, PAD)
    FULL = NP // _NS
    LAST = N - FULL * (_NS - 1)
    mesh = plsc.VectorSubcoreMesh(core_axis_name="c", subcore_axis_name="s")

    @functools.partial(
        pl.kernel,
        out_type=jax.ShapeDtypeStruct((_NC * N,), jnp.float32),
        mesh=mesh,
        scratch_types=[
            pltpu.VMEM_SHARED((NA,), jnp.float32),
            pltpu.VMEM((NCH, C), jnp.int32),
            pltpu.VMEM((C,), jnp.float32),
            pltpu.VMEM((FULL,), jnp.float32),
            pltpu.SemaphoreType.DMA,
            pltpu.SemaphoreType.DMA,
        ])
    def deg(dst_hbm, deg_hbm, dacc, dstv, ones, dzero, dsem, ssem):
        c = lax.axis_index("c")
        s = lax.axis_index("s")
        wid = s * _NC + c
        zv = jnp.zeros((_L,), jnp.float32)
        dslab = pltpu.make_async_copy(dst_hbm.at[wid], dstv, dsem)
        dslab.start()
        ov = jnp.full((_L,), 1.0, jnp.float32)
        for i in range(C // _L):
            ones[pl.ds(i * _L, _L)] = ov
        def dzrow(i, carry):
            dzero[pl.ds(i * _L, _L)] = zv
            return carry
        lax.fori_loop(0, FULL // _L, dzrow, 0)
        @pl.when(s < _NS - 1)
        def _():
            pltpu.sync_copy(dzero, dacc.at[pl.ds(s * FULL, FULL)])
        @pl.when(s == _NS - 1)
        def _():
            pltpu.sync_copy(dzero.at[pl.ds(0, LAST)],
                            dacc.at[pl.ds(s * FULL, LAST)])
        dslab.wait()
        plsc.subcore_barrier()

        B = 5
        def grp(gg, carry):
            g0 = gg * B
            for j in range(B):
                pltpu.async_copy(ones, dacc.at[dstv.at[g0 + j]], ssem,
                                 add=True)
            for j in range(B):
                pltpu.make_async_copy(ones, dacc.at[dstv.at[g0 + j]],
                                      ssem).wait()
            return carry
        lax.fori_loop(0, NCH // B, grp, 0)
        for g in range(NCH - NCH % B, NCH):
            pltpu.sync_copy(ones, dacc.at[dstv.at[g]], add=True)

        plsc.subcore_barrier()
        @pl.when(s < _NS - 1)
        def _():
            pltpu.sync_copy(dacc.at[pl.ds(s * FULL, FULL)], dzero)
            pltpu.sync_copy(dzero, deg_hbm.at[pl.ds(c * N + s * FULL, FULL)])
        @pl.when(s == _NS - 1)
        def _():
            pltpu.sync_copy(dacc.at[pl.ds(s * FULL, LAST)],
                            dzero.at[pl.ds(0, LAST)])
            pltpu.sync_copy(dzero.at[pl.ds(0, LAST)],
                            deg_hbm.at[pl.ds(c * N + s * FULL, LAST)])

    return deg


def _proj(x, Wl, Wr, bl):
    N, Din = x.shape
    Do = Wl.shape[1]
    BN = 2000

    def body(x_ref, wl_ref, wr_ref, b_ref, y_ref, z_ref):
        xb = x_ref[...]
        y_ref[...] = jnp.dot(xb, wl_ref[...],
                             preferred_element_type=jnp.float32)
        z_ref[...] = jnp.dot(xb, wr_ref[...],
                             preferred_element_type=jnp.float32) + b_ref[...]

    y, z = pl.pallas_call(
        body,
        grid=(N // BN,),
        in_specs=[
            pl.BlockSpec((BN, Din), lambda i: (i, 0)),
            pl.BlockSpec((Din, Do), lambda i: (0, 0)),
            pl.BlockSpec((Din, Do), lambda i: (0, 0)),
            pl.BlockSpec((1, Do), lambda i: (0, 0)),
        ],
        out_specs=[
            pl.BlockSpec((BN, Do), lambda i: (i, 0)),
            pl.BlockSpec((BN, Do), lambda i: (i, 0)),
        ],
        out_shape=[jax.ShapeDtypeStruct((N, Do), jnp.float32)] * 2,
    )(x, Wl, Wr, bl.reshape(1, -1))
    return y, z


def _comb_proj(p0, p1, z, d0, d1, Wl, Wr, bl, emit_h):
    N, Dh = z.shape
    Do = Wr.shape[1]
    BN = 2000

    def body(p0_ref, p1_ref, z_ref, d0_ref, d1_ref, wl_ref, wr_ref, b_ref,
             y_ref, z2_ref):
        deg = jnp.maximum(d0_ref[...] + d1_ref[...], 1.0)
        h = jnp.maximum((p0_ref[...] + p1_ref[...]) / deg + z_ref[...], 0.0)
        if emit_h:
            y_ref[...] = h
        else:
            y_ref[...] = jnp.dot(h, wl_ref[...],
                                 preferred_element_type=jnp.float32)
        z2_ref[...] = jnp.dot(h, wr_ref[...],
                              preferred_element_type=jnp.float32) + b_ref[...]

    return pl.pallas_call(
        body,
        grid=(N // BN,),
        in_specs=[
            pl.BlockSpec((BN, Dh), lambda i: (i, 0)),
            pl.BlockSpec((BN, Dh), lambda i: (i, 0)),
            pl.BlockSpec((BN, Dh), lambda i: (i, 0)),
            pl.BlockSpec((BN, 1), lambda i: (i, 0)),
            pl.BlockSpec((BN, 1), lambda i: (i, 0)),
            pl.BlockSpec(Wl.shape, lambda i: (0, 0)),
            pl.BlockSpec((Dh, Do), lambda i: (0, 0)),
            pl.BlockSpec((1, Do), lambda i: (0, 0)),
        ],
        out_specs=[
            pl.BlockSpec((BN, Dh), lambda i: (i, 0)),
            pl.BlockSpec((BN, Do), lambda i: (i, 0)),
        ],
        out_shape=[jax.ShapeDtypeStruct((N, Dh), jnp.float32),
                   jax.ShapeDtypeStruct((N, Do), jnp.float32)],
    )(p0, p1, z, d0, d1, Wl, Wr, bl.reshape(1, -1))


def _final(p0, p1, z2, d0, d1, Wl):
    N, Dh = p0.shape
    Do = Wl.shape[1]
    BN = 2000

    def body(p0_ref, p1_ref, z2_ref, d0_ref, d1_ref, wl_ref, o_ref):
        deg = jnp.maximum(d0_ref[...] + d1_ref[...], 1.0)
        m = (p0_ref[...] + p1_ref[...]) / deg
        u = (jnp.dot(m, wl_ref[...], preferred_element_type=jnp.float32)
             + z2_ref[...])
        mx = jnp.max(u, axis=1, keepdims=True)
        e = u - mx
        o_ref[...] = e - jnp.log(jnp.sum(jnp.exp(e), axis=1, keepdims=True))

    return pl.pallas_call(
        body,
        grid=(N // BN,),
        in_specs=[
            pl.BlockSpec((BN, Dh), lambda i: (i, 0)),
            pl.BlockSpec((BN, Dh), lambda i: (i, 0)),
            pl.BlockSpec((BN, Do), lambda i: (i, 0)),
            pl.BlockSpec((BN, 1), lambda i: (i, 0)),
            pl.BlockSpec((BN, 1), lambda i: (i, 0)),
            pl.BlockSpec((Dh, Do), lambda i: (0, 0)),
        ],
        out_specs=pl.BlockSpec((BN, Do), lambda i: (i, 0)),
        out_shape=jax.ShapeDtypeStruct((N, Do), jnp.float32),
    )(p0, p1, z2, d0, d1, Wl)


def kernel(x, edge_index, Wl0, bl0, Wr0, Wl1, bl1, Wr1, Wl2, bl2, Wr2):
    N, Din = x.shape
    E = edge_index.shape[1]
    Dh = Wl0.shape[1]
    NP = ((N + 128 * _NS - 1) // (128 * _NS)) * (128 * _NS)
    NW = _NC * _NS
    C = 80
    EPW_r = E // NW
    EPW = -(-EPW_r // C) * C
    pad = EPW - EPW_r
    src = jnp.pad(edge_index[0].reshape(NW, EPW_r),
                  ((0, 0), (0, pad))).reshape(-1)
    dump = jnp.broadcast_to(N + jnp.arange(pad, dtype=jnp.int32),
                            (NW, pad))
    dst = jnp.concatenate(
        [edge_index[1].reshape(NW, EPW_r), dump],
        axis=1).reshape(NW, EPW // C, C)

    agg_h = _make_agg(N, Dh, E, False, NP)

    degf = _make_deg(N, E, NP)(dst)
    y, z = _proj(x, Wl0, Wr0, bl0)
    (p,) = agg_h(y, src, dst)
    degp = degf.reshape(_NC, N)
    d0 = degp[0].reshape(N, 1)
    d1 = degp[1].reshape(N, 1)

    y, z = _comb_proj(p[0], p[1], z, d0, d1, Wl1, Wr1, bl1, emit_h=False)
    (p,) = agg_h(y, src, dst)

    h2, z2 = _comb_proj(p[0], p[1], z, d0, d1, Wl2, Wr2, bl2, emit_h=True)
    (p,) = agg_h(h2, src, dst)
    return _final(p[0], p[1], z2, d0, d1, Wl2)
